# baseline (speedup 1.0000x reference)
import functools

import jax
import jax.numpy as jnp
from jax import lax
from jax.experimental import pallas as pl
from jax.experimental.pallas import tpu as pltpu
from jax.experimental.pallas import tpu_sc as plsc

N, T = 4096, 200
D = 64
B = N * T
NC, NS = 2, 16
NW = NC * NS
K = 128
ROWS_PER_W = B // NW
STEPS = ROWS_PER_W // K


@functools.partial(
    pl.kernel,
    mesh=plsc.VectorSubcoreMesh(core_axis_name="c", subcore_axis_name="s"),
    out_type=jax.ShapeDtypeStruct((B, D), jnp.float32),
    compiler_params=pltpu.CompilerParams(use_tc_tiling_on_sc=False),
    scratch_types=[
        pltpu.VMEM((STEPS, K), jnp.int32),
        pltpu.VMEM((K, D), jnp.float32),
        pltpu.SemaphoreType.DMA,
    ],
)
def _gather_kernel(table_hbm, idx_hbm, out_hbm, idx_v, rows_v, sem):
    wid = lax.axis_index("s") * NC + lax.axis_index("c")
    pltpu.sync_copy(idx_hbm.at[pl.ds(wid * STEPS, STEPS)], idx_v)
    base = wid * ROWS_PER_W

    def step(g, carry):
        pltpu.async_copy(table_hbm.at[idx_v.at[g]], rows_v, sem).wait()
        pltpu.sync_copy(rows_v, out_hbm.at[pl.ds(base + g * K, K)])
        return carry

    lax.fori_loop(0, STEPS, step, 0)


def kernel(x, W_embed):
    idx = x.reshape(B // K, K).astype(jnp.int32)
    out = _gather_kernel(W_embed, idx)
    return out.reshape(N, T, D)


# speedup vs baseline: 1.1135x; 1.1135x over previous
import functools

import jax
import jax.numpy as jnp
from jax import lax
from jax.experimental import pallas as pl
from jax.experimental.pallas import tpu as pltpu
from jax.experimental.pallas import tpu_sc as plsc

N, T = 4096, 200
D = 64
B = N * T
NC, NS = 2, 16
NW = NC * NS
K = 128
ROWS_PER_W = B // NW
STEPS = ROWS_PER_W // K
NBUF = 4
NGRP = STEPS // NBUF


@functools.partial(
    pl.kernel,
    mesh=plsc.VectorSubcoreMesh(core_axis_name="c", subcore_axis_name="s"),
    out_type=jax.ShapeDtypeStruct((B, D), jnp.float32),
    compiler_params=pltpu.CompilerParams(use_tc_tiling_on_sc=False),
    scratch_types=(
        [pltpu.VMEM((STEPS, K), jnp.int32)]
        + [pltpu.VMEM((K, D), jnp.float32)] * NBUF
        + [pltpu.SemaphoreType.DMA] * (2 * NBUF)
    ),
)
def _gather_kernel(table_hbm, idx_hbm, out_hbm, idx_v, *scratch):
    rows = scratch[:NBUF]
    gsem = scratch[NBUF:2 * NBUF]
    wsem = scratch[2 * NBUF:]
    wid = lax.axis_index("s") * NC + lax.axis_index("c")
    pltpu.sync_copy(idx_hbm.at[pl.ds(wid * STEPS, STEPS)], idx_v)
    base = wid * ROWS_PER_W

    def start_gather(g, b):
        pltpu.async_copy(table_hbm.at[idx_v.at[g]], rows[b], gsem[b])

    def wait_gather(g, b):
        pltpu.make_async_copy(table_hbm.at[idx_v.at[g]], rows[b], gsem[b]).wait()

    def start_wb(g, b):
        pltpu.async_copy(rows[b], out_hbm.at[pl.ds(base + g * K, K)], wsem[b])

    def wait_wb(g, b):
        pltpu.make_async_copy(rows[b], out_hbm.at[pl.ds(base + g * K, K)], wsem[b]).wait()

    for b in range(NBUF):
        start_gather(b, b)

    def group(i, carry):
        g0 = i * NBUF
        for b in range(NBUF):
            wait_gather(g0 + b, b)
            start_wb(g0 + b, b)
        for b in range(NBUF):
            wait_wb(g0 + b, b)
            start_gather(g0 + NBUF + b, b)
        return carry

    lax.fori_loop(0, NGRP - 1, group, 0)

    g0 = (NGRP - 1) * NBUF
    for b in range(NBUF):
        wait_gather(g0 + b, b)
        start_wb(g0 + b, b)
    for b in range(NBUF):
        wait_wb(g0 + b, b)


def kernel(x, W_embed):
    idx = x.reshape(B // K, K).astype(jnp.int32)
    out = _gather_kernel(W_embed, idx)
    return out.reshape(N, T, D)
